# baseline (speedup 1.0000x reference)
import jax
import jax.numpy as jnp
from jax import lax
from jax.experimental import pallas as pl
from jax.experimental.pallas import tpu as pltpu
from jax.experimental.pallas import tpu_sc as plsc

N_LATENT = 128
N_GENES = 100000
N_OUT = 16
N_HIDDEN = 16
BATCH = 1024
G_OI = 2048

_NW = 32
_GPW = G_OI // _NW
_NCOLS = G_OI * N_OUT


def _sc_gather_body(genes_hbm, wtbl_hbm, rows_hbm, idx_v, rows_v, wsem):
    wid = lax.axis_index("s") * 2 + lax.axis_index("c")
    base = wid * _GPW
    pltpu.sync_copy(genes_hbm.at[pl.ds(base, _GPW)], idx_v)
    pltpu.async_copy(wtbl_hbm.at[idx_v], rows_v, wsem).wait()
    pltpu.sync_copy(rows_v, rows_hbm.at[pl.ds(base, _GPW)])


def _sc_gather(genes, wtbl256):
    mesh = plsc.VectorSubcoreMesh(core_axis_name="c", subcore_axis_name="s")
    return pl.kernel(
        _sc_gather_body,
        out_type=jax.ShapeDtypeStruct((G_OI, N_HIDDEN * N_OUT), jnp.float32),
        mesh=mesh,
        scratch_types=(
            pltpu.VMEM((_GPW,), jnp.int32),
            pltpu.VMEM((_GPW, N_HIDDEN * N_OUT), jnp.float32),
            pltpu.SemaphoreType.DMA,
        ),
    )(genes, wtbl256)


def _sc_bias_gather_body(genes_hbm, btbl_hbm, bg_hbm, idx_v, brows_v, bsem):
    wid = lax.axis_index("s") * 2 + lax.axis_index("c")
    base = wid * _GPW
    pltpu.sync_copy(genes_hbm.at[pl.ds(base, _GPW)], idx_v)
    pltpu.async_copy(btbl_hbm.at[idx_v], brows_v, bsem).wait()
    pltpu.sync_copy(brows_v, bg_hbm.at[pl.ds(base, _GPW)])


def _sc_bias_gather(genes, btbl):
    mesh = plsc.VectorSubcoreMesh(core_axis_name="c", subcore_axis_name="s")
    return pl.kernel(
        _sc_bias_gather_body,
        out_type=jax.ShapeDtypeStruct((G_OI, N_OUT), jnp.float32),
        mesh=mesh,
        scratch_types=(
            pltpu.VMEM((_GPW,), jnp.int32),
            pltpu.VMEM((_GPW, N_OUT), jnp.float32),
            pltpu.SemaphoreType.DMA,
        ),
        compiler_params=pltpu.CompilerParams(use_tc_tiling_on_sc=False),
    )(genes, btbl)


def _tc_transpose_body(rows_ref, bg_ref, wt9_ref, bt_ref):
    at = rows_ref[...].T
    for ot in range(2):
        for h in range(N_HIDDEN):
            s = h * 16 + ot * 8
            wt9_ref[ot, pl.ds(h * 8, 8), :] = at[s:s + 8, :]
    bt_ref[...] = bg_ref[...].T


def _tc_transpose(rows_all, bias_g):
    return pl.pallas_call(
        _tc_transpose_body,
        out_shape=(
            jax.ShapeDtypeStruct((2, 128, G_OI), jnp.float32),
            jax.ShapeDtypeStruct((N_OUT, G_OI), jnp.float32),
        ),
    )(rows_all, bias_g)


_BB = 256
_GT = 4


def _tc_matmul_body(latent_ref, w1_ref, b1_ref, scale_ref, shift_ref,
                    wt9_ref, bias_ref, out_ref):
    h = jnp.dot(latent_ref[...], w1_ref[...], preferred_element_type=jnp.float32)
    h = jnp.maximum(h + b1_ref[...], 0.0)
    h = h * scale_ref[...] + shift_ref[...]
    ii = lax.broadcasted_iota(jnp.int32, (N_HIDDEN, 128), 0)
    kk = lax.broadcasted_iota(jnp.int32, (N_HIDDEN, 128), 1) // 8
    expand = jnp.where(ii == kk, 1.0, 0.0).astype(jnp.float32)
    hrep = jnp.dot(h, expand, preferred_element_type=jnp.float32)
    h3 = jnp.broadcast_to(hrep[:, None, :], (_BB, 8, 128)).reshape(_BB * 8, 128)
    ri = lax.broadcasted_iota(jnp.int32, (_BB * 8, 128), 0) % 8
    ki = lax.broadcasted_iota(jnp.int32, (_BB * 8, 128), 1) % 8
    lhs = jnp.where(ri == ki, h3, 0.0)
    res = jnp.dot(lhs, wt9_ref[0], preferred_element_type=jnp.float32)
    bb = jnp.broadcast_to(bias_ref[0][None], (_BB, 8, _GT * 128))
    res = res + bb.reshape(_BB * 8, _GT * 128)
    for gt in range(_GT):
        out_ref[:, 0, gt] = res[:, gt * 128:(gt + 1) * 128].reshape(_BB, 8, 128)


def _tc_matmul(latent, w1, b1, scale, shift, wt9, bias9):
    grid = (BATCH // _BB, 2, N_OUT // 8 // 2 * G_OI // (128 * _GT))
    return pl.pallas_call(
        _tc_matmul_body,
        grid=grid,
        in_specs=[
            pl.BlockSpec((_BB, N_LATENT), lambda i, j, k: (i, 0)),
            pl.BlockSpec((N_LATENT, N_HIDDEN), lambda i, j, k: (0, 0)),
            pl.BlockSpec((1, N_HIDDEN), lambda i, j, k: (0, 0)),
            pl.BlockSpec((1, N_HIDDEN), lambda i, j, k: (0, 0)),
            pl.BlockSpec((1, N_HIDDEN), lambda i, j, k: (0, 0)),
            pl.BlockSpec((1, 128, _GT * 128), lambda i, j, k: (j, 0, k)),
            pl.BlockSpec((1, 8, _GT * 128), lambda i, j, k: (j, 0, k)),
        ],
        out_specs=pl.BlockSpec((_BB, 1, _GT, 8, 128),
                               lambda i, j, k: (i, j, k, 0, 0)),
        out_shape=jax.ShapeDtypeStruct((BATCH, 2, N_OUT, 8, 128), jnp.float32),
        compiler_params=pltpu.CompilerParams(
            dimension_semantics=("parallel", "parallel", "parallel"),
        ),
    )(latent, w1, b1, scale, shift, wt9, bias9)


def kernel(latent, genes_oi, W1, b1, gamma, beta, run_mean, run_var,
           weight_table, bias_table):
    genes = genes_oi.astype(jnp.int32)
    wtbl256 = weight_table.reshape(N_GENES, N_HIDDEN * N_OUT)

    rows_all = _sc_gather(genes, wtbl256)
    bias_g = _sc_bias_gather(genes, bias_table)

    wt9, bias_t = _tc_transpose(rows_all, bias_g)
    bias9 = bias_t.reshape(2, 8, G_OI)

    scale = (gamma / jnp.sqrt(run_var + 1e-5)).reshape(1, N_HIDDEN)
    shift = (beta - run_mean * scale[0]).reshape(1, N_HIDDEN)

    out5 = _tc_matmul(latent, W1, b1.reshape(1, N_HIDDEN), scale, shift,
                      wt9, bias9)
    out = out5.reshape(BATCH, 2, N_OUT, 8, 128).transpose(0, 2, 4, 1, 3)
    return out.reshape(BATCH, G_OI, N_OUT)


# speedup vs baseline: 1.1474x; 1.1474x over previous
import jax
import jax.numpy as jnp
from jax import lax
from jax.experimental import pallas as pl
from jax.experimental.pallas import tpu as pltpu
from jax.experimental.pallas import tpu_sc as plsc

N_LATENT = 128
N_GENES = 100000
N_OUT = 16
N_HIDDEN = 16
BATCH = 1024
G_OI = 2048

_NW = 32
_GPW = G_OI // _NW
_NCOLS = G_OI * N_OUT


def _sc_gather_body(genes_hbm, wtbl_hbm, bri_hbm, cmap_hbm, btbl_hbm,
                    rows_hbm, bg_hbm,
                    idx_v, bri_v, cmap_v, rows_v, brow_v, btile,
                    wsem, bsem):
    wid = lax.axis_index("s") * 2 + lax.axis_index("c")
    base = wid * _GPW
    pltpu.sync_copy(genes_hbm.at[pl.ds(base, _GPW)], idx_v)
    wcopy = pltpu.async_copy(wtbl_hbm.at[idx_v], rows_v, wsem)
    pltpu.sync_copy(bri_hbm.at[pl.ds(base, _GPW)], bri_v)
    bcopy = pltpu.async_copy(btbl_hbm.at[bri_v], brow_v, bsem)
    pltpu.sync_copy(cmap_hbm.at[pl.ds(base, _GPW)], cmap_v)

    bcopy.wait()
    iota16 = lax.iota(jnp.int32, 16)

    def extract_bias(g, carry):
        bbase = cmap_v[g, :] - iota16
        acc = jnp.zeros((16,), dtype=jnp.float32)
        for k in range(8):
            chunk = brow_v[g, pl.ds(k * N_OUT, N_OUT)]
            acc = jnp.where(bbase == k * N_OUT, chunk, acc)
        btile[g, :] = acc
        return carry

    lax.fori_loop(0, _GPW, extract_bias, 0, unroll=False)
    pltpu.sync_copy(btile, bg_hbm.at[pl.ds(base, _GPW)])
    wcopy.wait()
    pltpu.sync_copy(rows_v, rows_hbm.at[pl.ds(base, _GPW)])


def _sc_gather(genes, wtbl256, bri, cmap, btbl128):
    mesh = plsc.VectorSubcoreMesh(core_axis_name="c", subcore_axis_name="s")
    return pl.kernel(
        _sc_gather_body,
        out_type=(
            jax.ShapeDtypeStruct((G_OI, N_HIDDEN * N_OUT), jnp.float32),
            jax.ShapeDtypeStruct((G_OI, N_OUT), jnp.float32),
        ),
        mesh=mesh,
        scratch_types=(
            pltpu.VMEM((_GPW,), jnp.int32),
            pltpu.VMEM((_GPW,), jnp.int32),
            pltpu.VMEM((_GPW, N_OUT), jnp.int32),
            pltpu.VMEM((_GPW, N_HIDDEN * N_OUT), jnp.float32),
            pltpu.VMEM((_GPW, 128), jnp.float32),
            pltpu.VMEM((_GPW, N_OUT), jnp.float32),
            pltpu.SemaphoreType.DMA,
            pltpu.SemaphoreType.DMA,
        ),
    )(genes, wtbl256, bri, cmap, btbl128)


_RLB = 4096


def _tc_relayout_body(wt_ref, out_ref):
    out_ref[...] = wt_ref[...].T


def _tc_relayout(wt_native):
    return pl.pallas_call(
        _tc_relayout_body,
        grid=(pl.cdiv(N_GENES, _RLB),),
        in_specs=[pl.BlockSpec((N_HIDDEN * N_OUT, _RLB), lambda j: (0, j))],
        out_specs=pl.BlockSpec((_RLB, N_HIDDEN * N_OUT), lambda j: (j, 0)),
        out_shape=jax.ShapeDtypeStruct((N_GENES, N_HIDDEN * N_OUT),
                                       jnp.float32),
        compiler_params=pltpu.CompilerParams(
            dimension_semantics=("parallel",),
        ),
    )(wt_native)


def _tc_transpose_body(rows_ref, bg_ref, wt9_ref, bt_ref):
    at = rows_ref[...].T
    for ot in range(2):
        for h in range(N_HIDDEN):
            s = h * 16 + ot * 8
            wt9_ref[ot, pl.ds(h * 8, 8), :] = at[s:s + 8, :]
    bt_ref[...] = bg_ref[...].T


def _tc_transpose(rows_all, bias_g):
    return pl.pallas_call(
        _tc_transpose_body,
        out_shape=(
            jax.ShapeDtypeStruct((2, 128, G_OI), jnp.float32),
            jax.ShapeDtypeStruct((N_OUT, G_OI), jnp.float32),
        ),
    )(rows_all, bias_g)


_BB = 256
_GT = 4


def _tc_matmul_body(latent_ref, w1_ref, b1_ref, scale_ref, shift_ref,
                    wt9_ref, bias_ref, out_ref):
    h = jnp.dot(latent_ref[...], w1_ref[...], preferred_element_type=jnp.float32)
    h = jnp.maximum(h + b1_ref[...], 0.0)
    h = h * scale_ref[...] + shift_ref[...]
    ii = lax.broadcasted_iota(jnp.int32, (N_HIDDEN, 128), 0)
    kk = lax.broadcasted_iota(jnp.int32, (N_HIDDEN, 128), 1) // 8
    expand = jnp.where(ii == kk, 1.0, 0.0).astype(jnp.float32)
    hrep = jnp.dot(h, expand, preferred_element_type=jnp.float32)
    h3 = jnp.broadcast_to(hrep[:, None, :], (_BB, 8, 128)).reshape(_BB * 8, 128)
    ri = lax.broadcasted_iota(jnp.int32, (_BB * 8, 128), 0) % 8
    ki = lax.broadcasted_iota(jnp.int32, (_BB * 8, 128), 1) % 8
    lhs = jnp.where(ri == ki, h3, 0.0)
    res = jnp.dot(lhs, wt9_ref[0], preferred_element_type=jnp.float32)
    bb = jnp.broadcast_to(bias_ref[0][None], (_BB, 8, _GT * 128))
    res = res + bb.reshape(_BB * 8, _GT * 128)
    for gt in range(_GT):
        out_ref[:, 0, gt] = res[:, gt * 128:(gt + 1) * 128].reshape(_BB, 8, 128)


def _tc_matmul(latent, w1, b1, scale, shift, wt9, bias9):
    grid = (BATCH // _BB, 2, N_OUT // 8 // 2 * G_OI // (128 * _GT))
    return pl.pallas_call(
        _tc_matmul_body,
        grid=grid,
        in_specs=[
            pl.BlockSpec((_BB, N_LATENT), lambda i, j, k: (i, 0)),
            pl.BlockSpec((N_LATENT, N_HIDDEN), lambda i, j, k: (0, 0)),
            pl.BlockSpec((1, N_HIDDEN), lambda i, j, k: (0, 0)),
            pl.BlockSpec((1, N_HIDDEN), lambda i, j, k: (0, 0)),
            pl.BlockSpec((1, N_HIDDEN), lambda i, j, k: (0, 0)),
            pl.BlockSpec((1, 128, _GT * 128), lambda i, j, k: (j, 0, k)),
            pl.BlockSpec((1, 8, _GT * 128), lambda i, j, k: (j, 0, k)),
        ],
        out_specs=pl.BlockSpec((_BB, 1, _GT, 8, 128),
                               lambda i, j, k: (i, j, k, 0, 0)),
        out_shape=jax.ShapeDtypeStruct((BATCH, 2, N_OUT, 8, 128), jnp.float32),
        compiler_params=pltpu.CompilerParams(
            dimension_semantics=("parallel", "parallel", "parallel"),
        ),
    )(latent, w1, b1, scale, shift, wt9, bias9)


def kernel(latent, genes_oi, W1, b1, gamma, beta, run_mean, run_var,
           weight_table, bias_table):
    genes = genes_oi.astype(jnp.int32)
    wtbl256 = _tc_relayout(
        weight_table.transpose(1, 2, 0).reshape(N_HIDDEN * N_OUT, N_GENES))
    btbl128 = bias_table.reshape(N_GENES * N_OUT // 128, 128)
    bri = genes // 8
    cmap = ((genes % 8) * N_OUT)[:, None] + jnp.arange(N_OUT, dtype=jnp.int32)[None, :]

    rows_all, bias_g = _sc_gather(genes, wtbl256, bri, cmap, btbl128)

    wt9, bias_t = _tc_transpose(rows_all, bias_g)
    bias9 = bias_t.reshape(2, 8, G_OI)

    scale = (gamma / jnp.sqrt(run_var + 1e-5)).reshape(1, N_HIDDEN)
    shift = (beta - run_mean * scale[0]).reshape(1, N_HIDDEN)

    out5 = _tc_matmul(latent, W1, b1.reshape(1, N_HIDDEN), scale, shift,
                      wt9, bias9)
    out = out5.reshape(BATCH, 2, N_OUT, 8, 128).transpose(0, 2, 4, 1, 3)
    return out.reshape(BATCH, G_OI, N_OUT)


# speedup vs baseline: 1.2944x; 1.1281x over previous
import jax
import jax.numpy as jnp
from jax import lax
from jax.experimental import pallas as pl
from jax.experimental.pallas import tpu as pltpu
from jax.experimental.pallas import tpu_sc as plsc

N_LATENT = 128
N_GENES = 100000
N_OUT = 16
N_HIDDEN = 16
BATCH = 1024
G_OI = 2048

_NW = 32
_GPW = G_OI // _NW
_NCOLS = G_OI * N_OUT


def _sc_gather_body(genes_hbm, wtbl_hbm, rows_hbm, idx_v, rows_v, wsem):
    wid = lax.axis_index("s") * 2 + lax.axis_index("c")
    base = wid * _GPW
    pltpu.sync_copy(genes_hbm.at[pl.ds(base, _GPW)], idx_v)
    pltpu.async_copy(wtbl_hbm.at[idx_v], rows_v, wsem).wait()
    pltpu.sync_copy(rows_v, rows_hbm.at[pl.ds(base, _GPW)])


def _sc_gather(genes, wtbl384):
    mesh = plsc.VectorSubcoreMesh(core_axis_name="c", subcore_axis_name="s")
    return pl.kernel(
        _sc_gather_body,
        out_type=jax.ShapeDtypeStruct((G_OI, 384), jnp.float32),
        mesh=mesh,
        scratch_types=(
            pltpu.VMEM((_GPW,), jnp.int32),
            pltpu.VMEM((_GPW, 384), jnp.float32),
            pltpu.SemaphoreType.DMA,
        ),
    )(genes, wtbl384)


_RLB = 4096
_CW = 384


def _tc_relayout_body(wt_ref, bt_ref, out_ref):
    out_ref[:, 0:N_HIDDEN * N_OUT] = wt_ref[...].T
    out_ref[:, N_HIDDEN * N_OUT:N_HIDDEN * N_OUT + N_OUT] = bt_ref[...].T


def _tc_relayout(wt_native, bt_native):
    return pl.pallas_call(
        _tc_relayout_body,
        grid=(pl.cdiv(N_GENES, _RLB),),
        in_specs=[
            pl.BlockSpec((N_HIDDEN * N_OUT, _RLB), lambda j: (0, j)),
            pl.BlockSpec((N_OUT, _RLB), lambda j: (0, j)),
        ],
        out_specs=pl.BlockSpec((_RLB, _CW), lambda j: (j, 0)),
        out_shape=jax.ShapeDtypeStruct((N_GENES, _CW), jnp.float32),
        compiler_params=pltpu.CompilerParams(
            dimension_semantics=("parallel",),
        ),
    )(wt_native, bt_native)


def _tc_transpose_body(rows_ref, wt9_ref, bt_ref):
    a = rows_ref[...]
    at = a[:, 0:N_HIDDEN * N_OUT].T
    for ot in range(2):
        for h in range(N_HIDDEN):
            s = h * 16 + ot * 8
            wt9_ref[ot, pl.ds(h * 8, 8), :] = at[s:s + 8, :]
    bt_ref[...] = a[:, N_HIDDEN * N_OUT:N_HIDDEN * N_OUT + N_OUT].T


def _tc_transpose(rows_all):
    return pl.pallas_call(
        _tc_transpose_body,
        out_shape=(
            jax.ShapeDtypeStruct((2, 128, G_OI), jnp.float32),
            jax.ShapeDtypeStruct((N_OUT, G_OI), jnp.float32),
        ),
    )(rows_all)


_BB = 256
_GT = 4


def _tc_matmul_body(latent_ref, w1_ref, b1_ref, scale_ref, shift_ref,
                    wt9_ref, bias_ref, out_ref):
    h = jnp.dot(latent_ref[...], w1_ref[...], preferred_element_type=jnp.float32)
    h = jnp.maximum(h + b1_ref[...], 0.0)
    h = h * scale_ref[...] + shift_ref[...]
    ii = lax.broadcasted_iota(jnp.int32, (N_HIDDEN, 128), 0)
    kk = lax.broadcasted_iota(jnp.int32, (N_HIDDEN, 128), 1) // 8
    expand = jnp.where(ii == kk, 1.0, 0.0).astype(jnp.float32)
    hrep = jnp.dot(h, expand, preferred_element_type=jnp.float32)
    h3 = jnp.broadcast_to(hrep[:, None, :], (_BB, 8, 128)).reshape(_BB * 8, 128)
    ri = lax.broadcasted_iota(jnp.int32, (_BB * 8, 128), 0) % 8
    ki = lax.broadcasted_iota(jnp.int32, (_BB * 8, 128), 1) % 8
    lhs = jnp.where(ri == ki, h3, 0.0)
    res = jnp.dot(lhs, wt9_ref[0], preferred_element_type=jnp.float32)
    bb = jnp.broadcast_to(bias_ref[0][None], (_BB, 8, _GT * 128))
    res = res + bb.reshape(_BB * 8, _GT * 128)
    for gt in range(_GT):
        out_ref[:, 0, gt] = res[:, gt * 128:(gt + 1) * 128].reshape(_BB, 8, 128)


def _tc_matmul(latent, w1, b1, scale, shift, wt9, bias9):
    grid = (BATCH // _BB, 2, N_OUT // 8 // 2 * G_OI // (128 * _GT))
    return pl.pallas_call(
        _tc_matmul_body,
        grid=grid,
        in_specs=[
            pl.BlockSpec((_BB, N_LATENT), lambda i, j, k: (i, 0)),
            pl.BlockSpec((N_LATENT, N_HIDDEN), lambda i, j, k: (0, 0)),
            pl.BlockSpec((1, N_HIDDEN), lambda i, j, k: (0, 0)),
            pl.BlockSpec((1, N_HIDDEN), lambda i, j, k: (0, 0)),
            pl.BlockSpec((1, N_HIDDEN), lambda i, j, k: (0, 0)),
            pl.BlockSpec((1, 128, _GT * 128), lambda i, j, k: (j, 0, k)),
            pl.BlockSpec((1, 8, _GT * 128), lambda i, j, k: (j, 0, k)),
        ],
        out_specs=pl.BlockSpec((_BB, 1, _GT, 8, 128),
                               lambda i, j, k: (i, j, k, 0, 0)),
        out_shape=jax.ShapeDtypeStruct((BATCH, 2, N_OUT, 8, 128), jnp.float32),
        compiler_params=pltpu.CompilerParams(
            dimension_semantics=("parallel", "parallel", "parallel"),
        ),
    )(latent, w1, b1, scale, shift, wt9, bias9)


def kernel(latent, genes_oi, W1, b1, gamma, beta, run_mean, run_var,
           weight_table, bias_table):
    genes = genes_oi.astype(jnp.int32)
    wtbl384 = _tc_relayout(
        weight_table.transpose(1, 2, 0).reshape(N_HIDDEN * N_OUT, N_GENES),
        bias_table.T)

    rows_all = _sc_gather(genes, wtbl384)

    wt9, bias_t = _tc_transpose(rows_all)
    bias9 = bias_t.reshape(2, 8, G_OI)

    scale = (gamma / jnp.sqrt(run_var + 1e-5)).reshape(1, N_HIDDEN)
    shift = (beta - run_mean * scale[0]).reshape(1, N_HIDDEN)

    out5 = _tc_matmul(latent, W1, b1.reshape(1, N_HIDDEN), scale, shift,
                      wt9, bias9)
    out = out5.reshape(BATCH, 2, N_OUT, 8, 128).transpose(0, 2, 4, 1, 3)
    return out.reshape(BATCH, G_OI, N_OUT)


# speedup vs baseline: 1.3732x; 1.0609x over previous
import jax
import jax.numpy as jnp
from jax import lax
from jax.experimental import pallas as pl
from jax.experimental.pallas import tpu as pltpu
from jax.experimental.pallas import tpu_sc as plsc

N_LATENT = 128
N_GENES = 100000
N_OUT = 16
N_HIDDEN = 16
BATCH = 1024
G_OI = 2048

_NW = 32
_GPW = G_OI // _NW
_NCOLS = G_OI * N_OUT


def _sc_gather_body(genes_hbm, wtbl_hbm, rows_hbm, idx_v, rows_v, wsem):
    wid = lax.axis_index("s") * 2 + lax.axis_index("c")
    base = wid * _GPW
    pltpu.sync_copy(genes_hbm.at[pl.ds(base, _GPW)], idx_v)
    pltpu.async_copy(wtbl_hbm.at[idx_v], rows_v, wsem).wait()
    pltpu.sync_copy(rows_v, rows_hbm.at[pl.ds(base, _GPW)])


def _sc_gather(genes, wtbl384):
    mesh = plsc.VectorSubcoreMesh(core_axis_name="c", subcore_axis_name="s")
    return pl.kernel(
        _sc_gather_body,
        out_type=jax.ShapeDtypeStruct((G_OI, 384), jnp.float32),
        mesh=mesh,
        scratch_types=(
            pltpu.VMEM((_GPW,), jnp.int32),
            pltpu.VMEM((_GPW, 384), jnp.float32),
            pltpu.SemaphoreType.DMA,
        ),
    )(genes, wtbl384)


_RLB = 4096
_CW = 384


def _tc_relayout_body(wt_ref, bt_ref, out_ref):
    out_ref[:, 0:N_HIDDEN * N_OUT] = wt_ref[...].T
    out_ref[:, N_HIDDEN * N_OUT:N_HIDDEN * N_OUT + N_OUT] = bt_ref[...].T


def _tc_relayout(wt_native, bt_native):
    return pl.pallas_call(
        _tc_relayout_body,
        grid=(pl.cdiv(N_GENES, _RLB),),
        in_specs=[
            pl.BlockSpec((N_HIDDEN * N_OUT, _RLB), lambda j: (0, j)),
            pl.BlockSpec((N_OUT, _RLB), lambda j: (0, j)),
        ],
        out_specs=pl.BlockSpec((_RLB, _CW), lambda j: (j, 0)),
        out_shape=jax.ShapeDtypeStruct((N_GENES, _CW), jnp.float32),
        compiler_params=pltpu.CompilerParams(
            dimension_semantics=("parallel",),
        ),
    )(wt_native, bt_native)


def _tc_transpose_body(rows_ref, wt9_ref, bt_ref):
    a = rows_ref[...]
    at = a[:, 0:N_HIDDEN * N_OUT].T
    for ot in range(2):
        for h in range(N_HIDDEN):
            s = h * 16 + ot * 8
            wt9_ref[ot, pl.ds(h * 8, 8), :] = at[s:s + 8, :]
    bt_ref[...] = a[:, N_HIDDEN * N_OUT:N_HIDDEN * N_OUT + N_OUT].T


def _tc_transpose(rows_all):
    return pl.pallas_call(
        _tc_transpose_body,
        out_shape=(
            jax.ShapeDtypeStruct((2, 128, G_OI), jnp.float32),
            jax.ShapeDtypeStruct((N_OUT, G_OI), jnp.float32),
        ),
    )(rows_all)


_BB = 256
_GT = 8


def _tc_matmul_body(latent_ref, w1_ref, b1_ref, scale_ref, shift_ref,
                    wt9_ref, bias_ref, out_ref):
    h = jnp.dot(latent_ref[...], w1_ref[...], preferred_element_type=jnp.float32)
    h = jnp.maximum(h + b1_ref[...], 0.0)
    h = h * scale_ref[...] + shift_ref[...]
    ii = lax.broadcasted_iota(jnp.int32, (N_HIDDEN, 128), 0)
    kk = lax.broadcasted_iota(jnp.int32, (N_HIDDEN, 128), 1) // 8
    expand = jnp.where(ii == kk, 1.0, 0.0).astype(jnp.float32)
    hrep = jnp.dot(h, expand, preferred_element_type=jnp.float32)
    h3 = jnp.broadcast_to(hrep[:, None, :], (_BB, 8, 128)).reshape(_BB * 8, 128)
    ri = lax.broadcasted_iota(jnp.int32, (_BB * 8, 128), 0) % 8
    ki = lax.broadcasted_iota(jnp.int32, (_BB * 8, 128), 1) % 8
    lhs = jnp.where(ri == ki, h3, 0.0)
    res = jnp.dot(lhs, wt9_ref[0], preferred_element_type=jnp.float32)
    bb = jnp.broadcast_to(bias_ref[0][None], (_BB, 8, _GT * 128))
    res = res + bb.reshape(_BB * 8, _GT * 128)
    for gt in range(_GT):
        out_ref[:, 0, gt] = res[:, gt * 128:(gt + 1) * 128].reshape(_BB, 8, 128)


def _tc_matmul(latent, w1, b1, scale, shift, wt9, bias9):
    grid = (BATCH // _BB, 2, N_OUT // 8 // 2 * G_OI // (128 * _GT))
    return pl.pallas_call(
        _tc_matmul_body,
        grid=grid,
        in_specs=[
            pl.BlockSpec((_BB, N_LATENT), lambda i, j, k: (i, 0)),
            pl.BlockSpec((N_LATENT, N_HIDDEN), lambda i, j, k: (0, 0)),
            pl.BlockSpec((1, N_HIDDEN), lambda i, j, k: (0, 0)),
            pl.BlockSpec((1, N_HIDDEN), lambda i, j, k: (0, 0)),
            pl.BlockSpec((1, N_HIDDEN), lambda i, j, k: (0, 0)),
            pl.BlockSpec((1, 128, _GT * 128), lambda i, j, k: (j, 0, k)),
            pl.BlockSpec((1, 8, _GT * 128), lambda i, j, k: (j, 0, k)),
        ],
        out_specs=pl.BlockSpec((_BB, 1, _GT, 8, 128),
                               lambda i, j, k: (i, j, k, 0, 0)),
        out_shape=jax.ShapeDtypeStruct((BATCH, 2, N_OUT, 8, 128), jnp.float32),
        compiler_params=pltpu.CompilerParams(
            dimension_semantics=("parallel", "parallel", "parallel"),
        ),
    )(latent, w1, b1, scale, shift, wt9, bias9)


def kernel(latent, genes_oi, W1, b1, gamma, beta, run_mean, run_var,
           weight_table, bias_table):
    genes = genes_oi.astype(jnp.int32)
    wtbl384 = _tc_relayout(
        weight_table.transpose(1, 2, 0).reshape(N_HIDDEN * N_OUT, N_GENES),
        bias_table.T)

    rows_all = _sc_gather(genes, wtbl384)

    wt9, bias_t = _tc_transpose(rows_all)
    bias9 = bias_t.reshape(2, 8, G_OI)

    scale = (gamma / jnp.sqrt(run_var + 1e-5)).reshape(1, N_HIDDEN)
    shift = (beta - run_mean * scale[0]).reshape(1, N_HIDDEN)

    out5 = _tc_matmul(latent, W1, b1.reshape(1, N_HIDDEN), scale, shift,
                      wt9, bias9)
    out = out5.reshape(BATCH, 2, N_OUT, 8, 128).transpose(0, 2, 4, 1, 3)
    return out.reshape(BATCH, G_OI, N_OUT)
